# baseline (speedup 1.0000x reference)
import functools

import jax
import jax.numpy as jnp
from jax import lax
from jax.experimental import pallas as pl
from jax.experimental.pallas import tpu as pltpu
from jax.experimental.pallas import tpu_sc as plsc

B = 4096
F = 26
V = 100000
D = 64
H = 512
O = 256

NC = 2
NS = 16
NW = NC * NS

N_ROWS = B * F
ROWS_PER_W = N_ROWS // NW
CHUNK = 128
NITER = ROWS_PER_W // CHUNK

_sc_mesh = plsc.VectorSubcoreMesh(core_axis_name="c", subcore_axis_name="s")


@functools.partial(
    pl.kernel,
    out_type=jax.ShapeDtypeStruct((N_ROWS, D), jnp.float32),
    mesh=_sc_mesh,
    scratch_types=[
        pltpu.VMEM((NITER, CHUNK), jnp.int32),
        pltpu.VMEM((2, CHUNK, D), jnp.float32),
        pltpu.SemaphoreType.DMA,
        pltpu.SemaphoreType.DMA,
    ],
    compiler_params=pltpu.CompilerParams(use_tc_tiling_on_sc=False),
)
def _sc_gather(tables_hbm, idx_hbm, out_hbm, idx_v, rows_v, gsem, wsem):
    wid = lax.axis_index("s") * NC + lax.axis_index("c")
    base = wid * ROWS_PER_W
    pltpu.sync_copy(idx_hbm.at[wid], idx_v)

    def gather_start(j, slot):
        return pltpu.async_copy(tables_hbm.at[idx_v.at[j]], rows_v.at[slot], gsem)

    gather_start(0, 0)

    def body(j, _):
        slot = lax.rem(j, 2)
        @pl.when(j + 1 < NITER)
        def _():
            gather_start(j + 1, 1 - slot)

        pltpu.make_async_copy(tables_hbm.at[idx_v.at[j]], rows_v.at[slot], gsem).wait()
        cp = pltpu.async_copy(rows_v.at[slot], out_hbm.at[pl.ds(base + j * CHUNK, CHUNK)], wsem)
        cp.wait()
        return 0

    lax.fori_loop(0, NITER, body, 0)


def _mlp_body(x_ref, w1_ref, b1_ref, w2_ref, b2_ref, o_ref):
    h = jnp.dot(x_ref[...], w1_ref[...], preferred_element_type=jnp.float32)
    h = jnp.maximum(h + b1_ref[...], 0.0)
    o_ref[...] = jnp.dot(h, w2_ref[...], preferred_element_type=jnp.float32) + b2_ref[...]


BB = 512


def _tc_mlp(x, W1, b1, W2, b2):
    grid = (B // BB,)
    return pl.pallas_call(
        _mlp_body,
        grid=grid,
        in_specs=[
            pl.BlockSpec((BB, F * D), lambda i: (i, 0)),
            pl.BlockSpec((F * D, H), lambda i: (0, 0)),
            pl.BlockSpec((1, H), lambda i: (0, 0)),
            pl.BlockSpec((H, O), lambda i: (0, 0)),
            pl.BlockSpec((1, O), lambda i: (0, 0)),
        ],
        out_specs=pl.BlockSpec((BB, O), lambda i: (i, 0)),
        out_shape=jax.ShapeDtypeStruct((B, O), jnp.float32),
    )(x, W1, b1, W2, b2)


def kernel(features, tables, W1, b1, W2, b2):
    tables_flat = tables.reshape(F * V, D)
    flat_idx = features.astype(jnp.int32) + (jnp.arange(F, dtype=jnp.int32) * V)[None, :]
    idx = flat_idx.reshape(NW, NITER, CHUNK)
    emb_flat = _sc_gather(tables_flat, idx)
    x = emb_flat.reshape(B, F * D)
    return _tc_mlp(x, W1, b1.reshape(1, H), W2, b2.reshape(1, O))


# speedup vs baseline: 2.8368x; 2.8368x over previous
import functools

import jax
import jax.numpy as jnp
from jax import lax
from jax.experimental import pallas as pl
from jax.experimental.pallas import tpu as pltpu
from jax.experimental.pallas import tpu_sc as plsc

B = 4096
F = 26
V = 100000
D = 64
H = 512
O = 256

NC = 2
NS = 16
NW = NC * NS
L = 16

ROWS = F * D
NG = ROWS // 8
W = 4096
NFULL = 24
SCOL = NFULL * W
SLEN = 1664
TCOL = SCOL + SLEN
TLEN = V - TCOL
NWIN = NFULL + 2
NVREG = B // L

_sc_mesh = plsc.VectorSubcoreMesh(core_axis_name="c", subcore_axis_name="s")

_IOTA = None


@functools.partial(
    pl.kernel,
    out_type=jax.ShapeDtypeStruct((NG, 8, B), jnp.float32),
    mesh=_sc_mesh,
    scratch_types=[
        pltpu.VMEM((1, 1, B), jnp.int32),
        pltpu.VMEM((2, 8, W), jnp.float32),
        pltpu.VMEM((1, 8, B + L), jnp.float32),
        pltpu.VMEM((B + 2 * L,), jnp.int32),
        pltpu.VMEM((B + 2 * L,), jnp.int32),
        pltpu.SMEM((32,), jnp.int32),
        pltpu.SemaphoreType.DMA,
        pltpu.SemaphoreType.DMA,
        pltpu.SemaphoreType.DMA,
        pltpu.SemaphoreType.DMA,
    ],
    compiler_params=pltpu.CompilerParams(needs_layout_passes=False),
)
def _sc_gather(tt_hbm, ft_hbm, tail_hbm, xt_hbm, idx_v, buf_v, stage_v, wv_v, wb_v,
               woff_s, isem, gsem0, gsem1, wsem):
    wid = lax.axis_index("s") * NC + lax.axis_index("c")
    g_start = jnp.where(wid < 16, 7 * wid, 6 * wid + 16)
    n_grp = jnp.where(wid < 16, 7, 6)
    iota = lax.iota(jnp.int32, L)

    def bucketize(f):
        pltpu.async_copy(ft_hbm.at[pl.ds(f, 1)], idx_v, isem).wait()

        def per_window(w, ptr):
            woff_s[w] = ptr

            def per_vreg(j, ptr):
                v = idx_v[0, 0, pl.ds(j * L, L)]
                wvid = lax.shift_right_logical(v, 12)
                wvid = jnp.where(v >= TCOL, NWIN - 1, wvid)
                m = wvid == w
                mi = m.astype(jnp.int32)
                rank = plsc.cumsum(mi) - mi
                pos = jnp.where(m, ptr + rank, B + L + iota)
                plsc.store_scatter(wv_v, [pos], v)
                b = iota + j * L
                plsc.store_scatter(wb_v, [pos], b)
                return ptr + jnp.sum(mi)

            return lax.fori_loop(0, NVREG, per_vreg, ptr, unroll=False)

        end = lax.fori_loop(0, NWIN, per_window, jnp.int32(0), unroll=False)
        woff_s[NWIN] = end
        wb_v[pl.ds(B, L)] = iota + B
        wv_v[pl.ds(B, L)] = jnp.full((L,), TCOL, dtype=jnp.int32)

    def chunk_start(g, col, size, slot, sem):
        return pltpu.async_copy(
            tt_hbm.at[pl.ds(g, 1), :, pl.ds(col, size)],
            buf_v.at[pl.ds(slot, 1), :, pl.ds(0, size)], sem)

    def chunk_wait(size, slot, sem):
        pltpu.make_async_copy(
            tt_hbm.at[pl.ds(0, 1), :, pl.ds(0, size)],
            buf_v.at[pl.ds(slot, 1), :, pl.ds(0, size)], sem).wait()

    def pluck_window(w, col, slot):
        p0 = woff_s[w]
        p1 = woff_s[w + 1]
        n_t = lax.div(p1 - p0 + (L - 1), L)

        def per_tile(t, _):
            ko = p0 + t * L
            v = wv_v[pl.ds(ko, L)]
            b = wb_v[pl.ds(ko, L)]
            dv = jnp.minimum(jnp.maximum(v - col, 0), W - 1)
            slotv = jnp.full((L,), slot, dtype=jnp.int32)
            zv = jnp.zeros((L,), dtype=jnp.int32)
            for s in range(8):
                sv = jnp.full((L,), s, dtype=jnp.int32)
                g = plsc.load_gather(buf_v, [slotv, sv, dv])
                plsc.store_scatter(stage_v, [zv, sv, b], g)
            return 0

        lax.fori_loop(0, n_t, per_tile, 0, unroll=False)

    def per_group(k, f_prev):
        g = g_start + k
        f = g // 8

        @pl.when(f != f_prev)
        def _():
            bucketize(f)

        @pl.when(k >= 1)
        def _():
            pltpu.make_async_copy(stage_v.at[:, :, pl.ds(0, B)], xt_hbm.at[pl.ds(0, 1)], wsem).wait()

        chunk_start(g, 0, W, 0, gsem0)

        def per_pair(p, _):
            chunk_start(g, (2 * p + 1) * W, W, 1, gsem1)
            chunk_wait(W, 0, gsem0)
            pluck_window(2 * p, 2 * p * W, 0)

            @pl.when(2 * p + 2 < NFULL)
            def _():
                chunk_start(g, (2 * p + 2) * W, W, 0, gsem0)

            chunk_wait(W, 1, gsem1)
            pluck_window(2 * p + 1, (2 * p + 1) * W, 1)
            return 0

        lax.fori_loop(0, NFULL // 2, per_pair, 0, unroll=False)

        chunk_start(g, SCOL, SLEN, 0, gsem0)
        chunk_wait(SLEN, 0, gsem0)
        pluck_window(NFULL, SCOL, 0)

        pltpu.async_copy(tail_hbm.at[pl.ds(g, 1)],
                         buf_v.at[pl.ds(1, 1), :, pl.ds(0, 128)], gsem1)
        pltpu.make_async_copy(tail_hbm.at[pl.ds(0, 1)],
                              buf_v.at[pl.ds(1, 1), :, pl.ds(0, 128)], gsem1).wait()
        pluck_window(NFULL + 1, TCOL, 1)

        pltpu.async_copy(stage_v.at[:, :, pl.ds(0, B)], xt_hbm.at[pl.ds(g, 1)], wsem)
        return f

    lax.fori_loop(0, n_grp, per_group, jnp.int32(-1), unroll=False)
    pltpu.make_async_copy(stage_v.at[:, :, pl.ds(0, B)], xt_hbm.at[pl.ds(0, 1)], wsem).wait()


def _mlp_body(xt_ref, w1_ref, b1_ref, w2_ref, b2_ref, o_ref):
    h = lax.dot_general(
        xt_ref[...], w1_ref[...],
        dimension_numbers=(((0,), (0,)), ((), ())),
        preferred_element_type=jnp.float32,
    )
    h = jnp.maximum(h + b1_ref[...], 0.0)
    o_ref[...] = jnp.dot(h, w2_ref[...], preferred_element_type=jnp.float32) + b2_ref[...]


BB = 512


def _tc_mlp(xt, W1, b1, W2, b2):
    return pl.pallas_call(
        _mlp_body,
        grid=(B // BB,),
        in_specs=[
            pl.BlockSpec((ROWS, BB), lambda i: (0, i)),
            pl.BlockSpec((ROWS, H), lambda i: (0, 0)),
            pl.BlockSpec((1, H), lambda i: (0, 0)),
            pl.BlockSpec((H, O), lambda i: (0, 0)),
            pl.BlockSpec((1, O), lambda i: (0, 0)),
        ],
        out_specs=pl.BlockSpec((BB, O), lambda i: (i, 0)),
        out_shape=jax.ShapeDtypeStruct((B, O), jnp.float32),
    )(xt, W1, b1, W2, b2)


def kernel(features, tables, W1, b1, W2, b2):
    ttf = tables.transpose(0, 2, 1)
    tt = ttf.reshape(NG, 8, V)
    tail = jnp.pad(ttf[:, :, TCOL:], ((0, 0), (0, 0), (0, 128 - TLEN))).reshape(NG, 8, 128)
    ft = features.T.astype(jnp.int32).reshape(F, 1, B)
    xt = _sc_gather(tt, ft, tail).reshape(ROWS, B)
    return _tc_mlp(xt, W1, b1.reshape(1, H), W2, b2.reshape(1, O))
